# baseline (speedup 1.0000x reference)
import functools

import jax
import jax.numpy as jnp
from jax import lax
from jax.experimental import pallas as pl
from jax.experimental.pallas import tpu as pltpu
from jax.experimental.pallas import tpu_sc as plsc

VOCAB = 100000
D = 64
B = 16384
SEQ = 200
NT = 20
S1 = 10

_info = plsc.get_sparse_core_info()
_NC = _info.num_cores
_NS = _info.num_subcores
_NW = _NC * _NS
_ROWS_PER_W = B // _NW
_G = 4
_CW = _G * SEQ
_IDX_CHUNK = 128
_NBUF = 2
_NCHUNK = _ROWS_PER_W // _G


@functools.partial(
    pl.kernel,
    mesh=plsc.VectorSubcoreMesh(core_axis_name="c", subcore_axis_name="s"),
    out_type=jax.ShapeDtypeStruct((B * SEQ, D), jnp.float32),
    compiler_params=pltpu.CompilerParams(use_tc_tiling_on_sc=False),
    scratch_types=[
        pltpu.VMEM((_NBUF, _CW), jnp.int32),
        pltpu.VMEM((_NBUF, _CW, D), jnp.float32),
        pltpu.SemaphoreType.DMA,
        pltpu.SemaphoreType.DMA,
        pltpu.SemaphoreType.DMA,
    ],
)
def _prompt_embed(tok_hbm, table_hbm, out_hbm, idx_v, gbuf, sem_g, sw0, sw1):
    wid = lax.axis_index("s") * _NC + lax.axis_index("c")
    base_off = wid * _ROWS_PER_W * SEQ

    iota = lax.iota(jnp.int32, 16)
    sem_w = (sw0, sw1)

    def process(c, b):
        off = base_off + c * _CW
        idx_b = idx_v.at[b]
        gbuf_b = gbuf.at[b]
        pltpu.sync_copy(tok_hbm.at[pl.ds(off, _CW)], idx_b)
        for r in range(_G):
            rb = r * SEQ
            w0 = ((rb + 1) // 16) * 16
            g0 = idx_b[pl.ds(w0, 16)]
            g1 = idx_b[pl.ds(w0 + 16, 16)]
            t21 = g1[rb + NT + 1 - (w0 + 16)]
            for w, g in ((w0, g0), (w0 + 16, g1)):
                s = (w - rb) + iota
                in_r = (s >= 1) & (s <= NT + 1)
                cval = VOCAB + jnp.where(s <= S1, s - 1, s - 2)
                new = jnp.where(in_r, jnp.where(s == S1 + 1, t21, cval), g)
                idx_b[pl.ds(w, 16)] = new
        @pl.when(c >= _NBUF)
        def _():
            pltpu.make_async_copy(
                gbuf_b, out_hbm.at[pl.ds(off, _CW)], sem_w[b]).wait()
        copies = []
        for k in range(0, _CW, _IDX_CHUNK):
            n = min(_IDX_CHUNK, _CW - k)
            copies.append(pltpu.async_copy(
                table_hbm.at[idx_b.at[pl.ds(k, n)]],
                gbuf_b.at[pl.ds(k, n)], sem_g))
        for cp in copies:
            cp.wait()
        pltpu.async_copy(gbuf_b, out_hbm.at[pl.ds(off, _CW)], sem_w[b])

    def pair_body(o, carry):
        for b in range(_NBUF):
            process(o * _NBUF + b, b)
        return carry

    lax.fori_loop(0, _NCHUNK // _NBUF, pair_body, 0)
    for b in range(_NBUF):
        pltpu.make_async_copy(
            gbuf.at[b], out_hbm.at[pl.ds(base_off, _CW)], sem_w[b]).wait()


def kernel(tokens, wte_weight, learned_embedding):
    table = jnp.concatenate([wte_weight, learned_embedding], axis=0)
    out = _prompt_embed(tokens.reshape(B * SEQ), table)
    return out.reshape(B, SEQ, D)


# speedup vs baseline: 1.0058x; 1.0058x over previous
import functools

import jax
import jax.numpy as jnp
from jax import lax
from jax.experimental import pallas as pl
from jax.experimental.pallas import tpu as pltpu
from jax.experimental.pallas import tpu_sc as plsc

VOCAB = 100000
D = 64
B = 16384
SEQ = 200
NT = 20
S1 = 10

_info = plsc.get_sparse_core_info()
_NC = _info.num_cores
_NS = _info.num_subcores
_NW = _NC * _NS
_ROWS_PER_W = B // _NW
_G = 4
_CW = _G * SEQ
_IDX_CHUNK = 128
_NCHUNK = _ROWS_PER_W // _G


@functools.partial(
    pl.kernel,
    mesh=plsc.VectorSubcoreMesh(core_axis_name="c", subcore_axis_name="s"),
    out_type=jax.ShapeDtypeStruct((B * SEQ, D), jnp.float32),
    compiler_params=pltpu.CompilerParams(use_tc_tiling_on_sc=False),
    scratch_types=[
        pltpu.VMEM((2, _CW), jnp.int32),
        pltpu.VMEM((2, _CW, D), jnp.float32),
        pltpu.VMEM((2 * NT, D), jnp.float32),
        pltpu.SemaphoreType.DMA,
        pltpu.SemaphoreType.DMA,
        pltpu.SemaphoreType.DMA,
        pltpu.SemaphoreType.DMA,
    ],
)
def _prompt_embed(tok_hbm, wte_hbm, le_hbm, out_hbm,
                  idx_v, gbuf, corr_v, sg0, sg1, sw0, sw1):
    wid = lax.axis_index("s") * _NC + lax.axis_index("c")
    base_off = wid * _ROWS_PER_W * SEQ

    iota = lax.iota(jnp.int32, 16)
    sem_g = (sg0, sg1)
    sem_w = (sw0, sw1)

    pltpu.sync_copy(le_hbm, corr_v.at[pl.ds(0, NT)])
    pltpu.sync_copy(wte_hbm.at[pl.ds(0, NT)], corr_v.at[pl.ds(NT, NT)])
    for row in range(NT):
        for k in range(0, D, 16):
            a = corr_v[row, pl.ds(k, 16)]
            bvec = corr_v[NT + row, pl.ds(k, 16)]
            corr_v[row, pl.ds(k, 16)] = a - bvec

    def gather_descriptors(b):
        idx_b = idx_v.at[b]
        gbuf_b = gbuf.at[b]
        descs = []
        for k in range(0, _CW, _IDX_CHUNK):
            n = min(_IDX_CHUNK, _CW - k)
            descs.append((table_slice(idx_b, k, n), gbuf_b.at[pl.ds(k, n)]))
        return descs

    def table_slice(idx_b, k, n):
        return wte_hbm.at[idx_b.at[pl.ds(k, n)]]

    def stage(c, b):
        off = base_off + c * _CW
        idx_b = idx_v.at[b]
        pltpu.sync_copy(tok_hbm.at[pl.ds(off, _CW)], idx_b)
        for r in range(_G):
            rb = r * SEQ
            w0 = ((rb + 1) // 16) * 16
            g0 = idx_b[pl.ds(w0, 16)]
            g1 = idx_b[pl.ds(w0 + 16, 16)]
            t21 = g1[rb + NT + 1 - (w0 + 16)]
            for w, g in ((w0, g0), (w0 + 16, g1)):
                s = (w - rb) + iota
                in_r = (s >= 1) & (s <= NT + 1)
                cval = jnp.where(s <= S1, s - 1, s - 2)
                new = jnp.where(in_r, jnp.where(s == S1 + 1, t21, cval), g)
                idx_b[pl.ds(w, 16)] = new
        @pl.when(c >= 2)
        def _():
            pltpu.make_async_copy(
                gbuf.at[b], out_hbm.at[pl.ds(off, _CW)], sem_w[b]).wait()
        for src, dst in gather_descriptors(b):
            pltpu.async_copy(src, dst, sem_g[b])

    def finish(c, b):
        off = base_off + c * _CW
        for src, dst in gather_descriptors(b):
            pltpu.make_async_copy(src, dst, sem_g[b]).wait()
        gbuf_b = gbuf.at[b]
        for r in range(_G):
            for j in range(NT):
                row = r * SEQ + 1 + j + (j >= S1)
                for k in range(0, D, 16):
                    gbuf_b[row, pl.ds(k, 16)] = (
                        gbuf_b[row, pl.ds(k, 16)] + corr_v[j, pl.ds(k, 16)])
        pltpu.async_copy(gbuf_b, out_hbm.at[pl.ds(off, _CW)], sem_w[b])

    stage(0, 0)

    def pair_body(o, carry):
        c = o * 2
        stage(c + 1, 1)
        finish(c, 0)
        stage(c + 2, 0)
        finish(c + 1, 1)
        return carry

    lax.fori_loop(0, _NCHUNK // 2 - 1, pair_body, 0)
    c = _NCHUNK - 2
    stage(c + 1, 1)
    finish(c, 0)
    finish(c + 1, 1)
    for b in range(2):
        pltpu.make_async_copy(
            gbuf.at[b], out_hbm.at[pl.ds(base_off, _CW)], sem_w[b]).wait()


def kernel(tokens, wte_weight, learned_embedding):
    out = _prompt_embed(tokens.reshape(B * SEQ), wte_weight,
                        learned_embedding)
    return out.reshape(B, SEQ, D)
